# baseline (speedup 1.0000x reference)
import functools

import jax
import jax.numpy as jnp
from jax import lax
from jax.experimental import pallas as pl
from jax.experimental.pallas import tpu as pltpu
from jax.experimental.pallas import tpu_sc as plsc

B = 4096
F = 26
D = 16
CHUNK = 128
ROWS = B * F // CHUNK


@functools.cache
def _make_sc_gather():
    info = plsc.get_sparse_core_info()
    nw = info.num_cores * info.num_subcores
    cpw = ROWS // nw
    mesh = plsc.VectorSubcoreMesh(core_axis_name="c", subcore_axis_name="s")

    @functools.partial(
        pl.kernel,
        mesh=mesh,
        compiler_params=pltpu.CompilerParams(use_tc_tiling_on_sc=False),
        out_type=jax.ShapeDtypeStruct((B * F, D), jnp.float32),
        scratch_types=[
            pltpu.VMEM((cpw, CHUNK), jnp.int32),
            pltpu.VMEM((cpw * CHUNK, D), jnp.float32),
            pltpu.SemaphoreType.DMA,
        ],
    )
    def sc_gather(idx_hbm, table_hbm, out_hbm, idx_v, rows_v, sem):
        wid = lax.axis_index("s") * info.num_cores + lax.axis_index("c")
        base = wid * cpw
        pltpu.sync_copy(idx_hbm.at[wid], idx_v)
        copies = []
        for c in range(cpw):
            copies.append(
                pltpu.async_copy(
                    table_hbm.at[idx_v.at[c]],
                    rows_v.at[pl.ds(c * CHUNK, CHUNK)],
                    sem,
                )
            )
        for cp in copies:
            cp.wait()
        pltpu.sync_copy(rows_v, out_hbm.at[pl.ds(base * CHUNK, cpw * CHUNK)])

    return sc_gather


def _tc_compute(rows2d, fv, W1, b1, W2, b2, W3, b3):
    BB = 512
    A = jnp.tile(jnp.eye(D, dtype=jnp.float32), (F, 1))
    R = jnp.repeat(jnp.eye(F, dtype=jnp.float32), D, axis=1)
    b1r = b1.reshape(1, -1)
    b2r = b2.reshape(1, -1)
    b3r = b3.reshape(1, -1)

    def body(rows_ref, fv_ref, R_ref, A_ref, W1_ref, b1_ref, W2_ref,
             b2_ref, W3_ref, b3_ref, out_ref):
        fvr = jnp.dot(fv_ref[...], R_ref[...], preferred_element_type=jnp.float32)
        fe = rows_ref[...] * fvr
        s = jnp.dot(fe, A_ref[...], preferred_element_type=jnp.float32)
        q = jnp.dot(fe * fe, A_ref[...], preferred_element_type=jnp.float32)
        bi = (s * s - q) * 0.5
        h = jnp.maximum(jnp.dot(bi, W1_ref[...], preferred_element_type=jnp.float32)
                        + b1_ref[...], 0.0)
        h = jnp.maximum(jnp.dot(h, W2_ref[...], preferred_element_type=jnp.float32)
                        + b2_ref[...], 0.0)
        o = jnp.dot(h, W3_ref[...], preferred_element_type=jnp.float32) + b3_ref[...]
        out_ref[...] = jax.nn.sigmoid(o)

    return pl.pallas_call(
        body,
        grid=(B // BB,),
        in_specs=[
            pl.BlockSpec((BB, F * D), lambda i: (i, 0)),
            pl.BlockSpec((BB, F), lambda i: (i, 0)),
            pl.BlockSpec((F, F * D), lambda i: (0, 0)),
            pl.BlockSpec((F * D, D), lambda i: (0, 0)),
            pl.BlockSpec((D, 32), lambda i: (0, 0)),
            pl.BlockSpec((1, 32), lambda i: (0, 0)),
            pl.BlockSpec((32, 32), lambda i: (0, 0)),
            pl.BlockSpec((1, 32), lambda i: (0, 0)),
            pl.BlockSpec((32, 1), lambda i: (0, 0)),
            pl.BlockSpec((1, 1), lambda i: (0, 0)),
        ],
        out_specs=pl.BlockSpec((BB, 1), lambda i: (i, 0)),
        out_shape=jax.ShapeDtypeStruct((B, 1), jnp.float32),
    )(rows2d, fv, R, A, W1, b1r, W2, b2r, W3, b3r)


def kernel(feat_index, feat_value, emb_table, W1, b1, W2, b2, W3, b3):
    nw = 32
    fidx = feat_index.astype(jnp.int32).reshape(nw, ROWS // nw, CHUNK)
    rows = _make_sc_gather()(fidx, emb_table)
    rows2d = rows.reshape(B, F * D)
    return _tc_compute(rows2d, feat_value, W1, b1, W2, b2, W3, b3)


# speedup vs baseline: 1.0385x; 1.0385x over previous
import functools

import jax
import jax.numpy as jnp
from jax import lax
from jax.experimental import pallas as pl
from jax.experimental.pallas import tpu as pltpu
from jax.experimental.pallas import tpu_sc as plsc

B = 4096
F = 26
D = 16
CHUNK = 128
NW = 32
EPW = B // NW
RPW = EPW * F
CPW = RPW // CHUNK
PACK = 128 // D


@functools.cache
def _make_sc_pool():
    mesh = plsc.VectorSubcoreMesh(core_axis_name="c", subcore_axis_name="s")

    @functools.partial(
        pl.kernel,
        mesh=mesh,
        compiler_params=pltpu.CompilerParams(use_tc_tiling_on_sc=False,
                                             needs_layout_passes=False),
        out_type=jax.ShapeDtypeStruct((B * D // 128, 128), jnp.float32),
        scratch_types=[
            pltpu.VMEM((CPW, CHUNK), jnp.int32),
            pltpu.VMEM((RPW,), jnp.float32),
            pltpu.VMEM((RPW, D), jnp.float32),
            pltpu.VMEM((EPW * D // 128, 128), jnp.float32),
            pltpu.SemaphoreType.DMA,
        ],
    )
    def sc_pool(idx_hbm, fv_hbm, table_hbm, out_hbm,
                idx_v, fv_v, rows_v, bi_v, sem):
        wid = lax.axis_index("s") * 2 + lax.axis_index("c")
        pltpu.sync_copy(idx_hbm.at[wid], idx_v)
        pltpu.sync_copy(fv_hbm.at[wid], fv_v)
        copies = []
        for c in range(CPW):
            copies.append(
                pltpu.async_copy(
                    table_hbm.at[idx_v.at[c]],
                    rows_v.at[pl.ds(c * CHUNK, CHUNK)],
                    sem,
                )
            )
        for cp in copies:
            cp.wait()

        def e_body(e, carry):
            s = jnp.zeros((D,), jnp.float32)
            q = jnp.zeros((D,), jnp.float32)
            for f in range(F):
                p = e * F + f
                row = rows_v[p]
                vb = plsc.load_gather(fv_v, [jnp.full((D,), p, jnp.int32)])
                ve = row * vb
                s = s + ve
                q = q + ve * ve
            bi = (s * s - q) * 0.5
            bi_v[e // PACK, pl.ds((e % PACK) * D, D)] = bi
            return carry

        lax.fori_loop(0, EPW, e_body, 0)
        pltpu.sync_copy(bi_v, out_hbm.at[pl.ds(wid * (EPW * D // 128),
                                               EPW * D // 128)])

    return sc_pool


def _tc_mlp(bi_p, W1, b1, W2, b2, W3, b3):
    eye = jnp.eye(PACK, dtype=jnp.float32)
    W1p = jnp.kron(eye, W1)
    W2p = jnp.kron(eye, W2)
    W3p = jnp.kron(eye, W3)
    b1p = jnp.tile(b1, PACK).reshape(1, -1)
    b2p = jnp.tile(b2, PACK).reshape(1, -1)
    b3p = jnp.tile(b3, PACK).reshape(1, -1)
    BR = B // PACK

    def body(bi_ref, W1_ref, b1_ref, W2_ref, b2_ref, W3_ref, b3_ref, out_ref):
        h = jnp.maximum(
            jnp.dot(bi_ref[...], W1_ref[...], preferred_element_type=jnp.float32)
            + b1_ref[...], 0.0)
        h = jnp.maximum(
            jnp.dot(h, W2_ref[...], preferred_element_type=jnp.float32)
            + b2_ref[...], 0.0)
        o = jnp.dot(h, W3_ref[...], preferred_element_type=jnp.float32) + b3_ref[...]
        out_ref[...] = jax.nn.sigmoid(o)

    out = pl.pallas_call(
        body,
        out_shape=jax.ShapeDtypeStruct((BR, PACK), jnp.float32),
    )(bi_p, W1p, b1p, W2p, b2p, W3p, b3p)
    return out.reshape(B, 1)


def kernel(feat_index, feat_value, emb_table, W1, b1, W2, b2, W3, b3):
    fidx = feat_index.astype(jnp.int32).reshape(NW, CPW, CHUNK)
    fv = feat_value.reshape(NW, RPW)
    bi_p = _make_sc_pool()(fidx, fv, emb_table)
    return _tc_mlp(bi_p, W1, b1, W2, b2, W3, b3)
